# baseline (speedup 1.0000x reference)
import functools
import math

import jax
import jax.numpy as jnp
from jax import lax
from jax.experimental import pallas as pl
from jax.experimental.pallas import tpu as pltpu
from jax.experimental.pallas import tpu_sc as plsc

_VOCAB = 1000000
_D = 64
_S = 200
_B = 4096

_NC = 2
_NS = 16
_NW = _NC * _NS
_CHUNK = 128
_ROWS_PER_W = _B * _S // _NW
_CPW = _ROWS_PER_W // _CHUNK
_POS_EXT = _S + _CHUNK
_LANES = 16
_DSLICES = _D // _LANES


def _emb_body(x_hbm, emb_hbm, pos_hbm, out_hbm,
              idx_v, pos_v, in_v, out_v, gsem0, gsem1, ssem0, ssem1):
    wid = lax.axis_index("s") * _NC + lax.axis_index("c")

    pltpu.sync_copy(x_hbm.at[wid], idx_v)
    pltpu.sync_copy(pos_hbm, pos_v)

    gsems = (gsem0, gsem1)
    ssems = (ssem0, ssem1)
    row_base = wid * _ROWS_PER_W

    def gather_start(c, b):
        pltpu.async_copy(emb_hbm.at[idx_v.at[c]], in_v.at[b], gsems[b])

    def gather_wait(b):
        pltpu.make_async_copy(emb_hbm.at[idx_v.at[0]], in_v.at[b], gsems[b]).wait()

    def store_start(c, b):
        pltpu.async_copy(out_v.at[b], out_hbm.at[pl.ds(row_base + c * _CHUNK, _CHUNK)],
                         ssems[b])

    def store_wait(b):
        pltpu.make_async_copy(out_v.at[b], out_hbm.at[pl.ds(row_base, _CHUNK)],
                              ssems[b]).wait()

    gather_start(0, 0)
    gather_start(1, 1)

    def chunk_step(c, b):
        gather_wait(b)
        @pl.when(c >= 2)
        def _():
            store_wait(b)
        p0 = (c * _CHUNK) % _S

        def row_step(j, carry):
            for d in range(_DSLICES):
                v = in_v[b, j, pl.ds(d * _LANES, _LANES)] * 8.0 \
                    + pos_v[p0 + j, pl.ds(d * _LANES, _LANES)]
                out_v[b, j, pl.ds(d * _LANES, _LANES)] = v
            return carry

        lax.fori_loop(0, _CHUNK, row_step, 0, unroll=2)
        store_start(c, b)
        @pl.when(c + 2 < _CPW)
        def _():
            gather_start(c + 2, b)

    def loop_body(c2, carry):
        chunk_step(2 * c2, 0)
        chunk_step(2 * c2 + 1, 1)
        return carry

    lax.fori_loop(0, _CPW // 2, loop_body, 0)
    store_wait(0)
    store_wait(1)


def kernel(x, emb_table, pos_table):
    xr = x.reshape(_NW, _CPW, _CHUNK)
    pos_ext = jnp.concatenate([pos_table, pos_table[:_CHUNK]], axis=0)

    mesh = plsc.VectorSubcoreMesh(core_axis_name="c", subcore_axis_name="s")
    run = functools.partial(
        pl.kernel,
        mesh=mesh,
        compiler_params=pltpu.CompilerParams(use_tc_tiling_on_sc=False),
        out_type=jax.ShapeDtypeStruct((_B * _S, _D), jnp.float32),
        scratch_types=[
            pltpu.VMEM((_CPW, _CHUNK), jnp.int32),
            pltpu.VMEM((_POS_EXT, _D), jnp.float32),
            pltpu.VMEM((2, _CHUNK, _D), jnp.float32),
            pltpu.VMEM((2, _CHUNK, _D), jnp.float32),
            pltpu.SemaphoreType.DMA,
            pltpu.SemaphoreType.DMA,
            pltpu.SemaphoreType.DMA,
            pltpu.SemaphoreType.DMA,
        ],
    )(_emb_body)
    out = run(xr, emb_table, pos_ext)
    return out.reshape(_B, _S, _D)
